# baseline (speedup 1.0000x reference)
import functools

import jax
import jax.numpy as jnp
from jax import lax
from jax.experimental import pallas as pl
from jax.experimental.pallas import tpu as pltpu
from jax.experimental.pallas import tpu_sc as plsc

N_NODES = 10000
N_EDGES = 320000
D_IN = 128
D_HID = 16
D_OUT = 5

NC = 2
NS = 16
NW = NC * NS
CH = 128
NBUF = 8
NINF = NBUF // 2
K = NBUF * (-(-N_EDGES // (NW * CH * NBUF)))
E_PAD = NW * K * CH
N_ACC = 10240
ZR = N_ACC // NS

_mesh = plsc.VectorSubcoreMesh(core_axis_name="c", subcore_axis_name="s")
_sc_params = pltpu.CompilerParams(use_tc_tiling_on_sc=False)


def _zero_accum(s, zbuf, accum):
  def zrow(i, _):
    zbuf[i, :] = jnp.zeros((D_HID,), jnp.float32)
    return ()
  lax.fori_loop(0, ZR, zrow, ())
  pltpu.sync_copy(zbuf, accum.at[pl.ds(s * ZR, ZR)])
  plsc.subcore_barrier()


def _flush_accum(c, s, accum, out_hbm):
  plsc.subcore_barrier()
  pltpu.sync_copy(accum.at[pl.ds(s * ZR, ZR)],
                  out_hbm.at[c, pl.ds(s * ZR, ZR)])


@functools.partial(
    pl.kernel,
    out_type=jax.ShapeDtypeStruct((NC, N_ACC, D_HID), jnp.float32),
    mesh=_mesh,
    compiler_params=_sc_params,
    scratch_types=[
        pltpu.VMEM((K, CH), jnp.int32),
        pltpu.VMEM((CH, D_HID), jnp.float32),
        pltpu.VMEM((ZR, D_HID), jnp.float32),
        pltpu.VMEM_SHARED((N_ACC, D_HID), jnp.float32),
    ],
)
def _sc_degree(dst_hbm, out_hbm, dst_v, ones_v, zbuf, accum):
  c = lax.axis_index("c")
  s = lax.axis_index("s")
  wid = s * NC + c
  _zero_accum(s, zbuf, accum)

  def orow(i, _):
    ones_v[i, :] = jnp.ones((D_HID,), jnp.float32)
    return ()
  lax.fori_loop(0, CH, orow, ())
  pltpu.sync_copy(dst_hbm.at[wid], dst_v)

  def body(j, _):
    pltpu.sync_copy(ones_v, accum.at[dst_v.at[j]], add=True)
    return ()
  lax.fori_loop(0, K, body, ())
  _flush_accum(c, s, accum, out_hbm)


@functools.partial(
    pl.kernel,
    out_type=jax.ShapeDtypeStruct((NC, N_ACC, D_HID), jnp.float32),
    mesh=_mesh,
    compiler_params=_sc_params,
    scratch_types=[
        pltpu.VMEM((K, CH), jnp.int32),
        pltpu.VMEM((K, CH), jnp.int32),
        [pltpu.VMEM((CH, D_HID), jnp.float32)] * NBUF,
        pltpu.VMEM((ZR, D_HID), jnp.float32),
        pltpu.VMEM_SHARED((N_ACC, D_HID), jnp.float32),
        [pltpu.SemaphoreType.DMA] * NBUF,
        [pltpu.SemaphoreType.DMA] * NBUF,
    ],
)
def _sc_aggregate(table_hbm, src_hbm, dst_hbm, out_hbm,
                  src_v, dst_v, rows, zbuf, accum, gsem, ssem):
  c = lax.axis_index("c")
  s = lax.axis_index("s")
  wid = s * NC + c
  _zero_accum(s, zbuf, accum)
  pltpu.sync_copy(src_hbm.at[wid], src_v)
  pltpu.sync_copy(dst_hbm.at[wid], dst_v)

  for r in range(NINF):
    pltpu.async_copy(table_hbm.at[src_v.at[r]], rows[r], gsem[r])

  def body(jj, _):
    for r in range(NBUF):
      j = jj * NBUF + r
      rd = (r + NINF) % NBUF
      pltpu.make_async_copy(table_hbm.at[src_v.at[j]], rows[r], gsem[r]).wait()
      pltpu.async_copy(rows[r], accum.at[dst_v.at[j]], ssem[r], add=True)

      @pl.when(j >= NINF)
      def _():
        pltpu.make_async_copy(rows[rd], accum.at[dst_v.at[j - NINF]],
                              ssem[rd]).wait()

      @pl.when(j + NINF < K)
      def _():
        pltpu.async_copy(table_hbm.at[src_v.at[j + NINF]], rows[rd], gsem[rd])
    return ()

  lax.fori_loop(0, K // NBUF, body, ())
  for i in range(K - NINF, K):
    pltpu.make_async_copy(rows[i % NBUF], accum.at[dst_v.at[i]],
                          ssem[i % NBUF]).wait()
  _flush_accum(c, s, accum, out_hbm)


def _tca_body(x_ref, w1_ref, out_ref):
  out_ref[...] = jnp.dot(x_ref[...], w1_ref[...],
                         preferred_element_type=jnp.float32)


def _tc1_body(deg_ref, xw1_ref, dinv_ref, p_ref):
  deg = deg_ref[0, :N_NODES, :] + deg_ref[1, :N_NODES, :] + 1.0
  dinv = lax.rsqrt(deg)
  dinv_ref[...] = dinv
  p_ref[...] = xw1_ref[...] * dinv


def _tc2_body(agg_ref, p_ref, dinv_ref, b1_ref, q_ref):
  dinv = dinv_ref[...]
  s = agg_ref[0, :N_NODES, :] + agg_ref[1, :N_NODES, :] + p_ref[...]
  h = jnp.maximum(dinv * s + b1_ref[...], 0.0)
  q_ref[...] = h * dinv


def _tc3_body(agg_ref, q_ref, dinv_ref, w2_ref, b2_ref, out_ref):
  s = agg_ref[0, :N_NODES, :] + agg_ref[1, :N_NODES, :] + q_ref[...]
  z = jnp.dot(dinv_ref[...] * s, w2_ref[...],
              preferred_element_type=jnp.float32) + b2_ref[...]
  m = jnp.max(z, axis=1, keepdims=True)
  zs = z - m
  out_ref[...] = zs - jnp.log(jnp.sum(jnp.exp(zs), axis=1, keepdims=True))


def kernel(x, edge_index, W1, b1, W2, b2):
  src = jnp.concatenate(
      [edge_index[0], jnp.zeros((E_PAD - N_EDGES,), jnp.int32)])
  dst = jnp.concatenate(
      [edge_index[1], jnp.full((E_PAD - N_EDGES,), N_NODES, jnp.int32)])
  src3 = src.reshape(NW, K, CH)
  dst3 = dst.reshape(NW, K, CH)

  deg16 = _sc_degree(dst3)

  xw1 = pl.pallas_call(
      _tca_body,
      out_shape=jax.ShapeDtypeStruct((N_NODES, D_HID), jnp.float32),
  )(x, W1)

  dinv, p = pl.pallas_call(
      _tc1_body,
      out_shape=(
          jax.ShapeDtypeStruct((N_NODES, D_HID), jnp.float32),
          jax.ShapeDtypeStruct((N_NODES, D_HID), jnp.float32),
      ),
  )(deg16, xw1)

  agg1 = _sc_aggregate(p, src3, dst3)

  q = pl.pallas_call(
      _tc2_body,
      out_shape=jax.ShapeDtypeStruct((N_NODES, D_HID), jnp.float32),
  )(agg1, p, dinv, b1.reshape(1, D_HID))

  agg2 = _sc_aggregate(q, src3, dst3)

  out = pl.pallas_call(
      _tc3_body,
      out_shape=jax.ShapeDtypeStruct((N_NODES, D_OUT), jnp.float32),
  )(agg2, q, dinv, W2, b2.reshape(1, D_OUT))
  return out


# speedup vs baseline: 1.0604x; 1.0604x over previous
import functools

import jax
import jax.numpy as jnp
from jax import lax
from jax.experimental import pallas as pl
from jax.experimental.pallas import tpu as pltpu
from jax.experimental.pallas import tpu_sc as plsc

N_NODES = 10000
N_EDGES = 320000
D_IN = 128
D_HID = 16
D_OUT = 5

NC = 2
NS = 16
NW = NC * NS
CH = 128
NBUF = 8
NINF = NBUF // 2
K = NBUF * (-(-N_EDGES // (NW * CH * NBUF)))
E_PAD = NW * K * CH
N_ACC = 10240
ZR = N_ACC // NS

_mesh = plsc.VectorSubcoreMesh(core_axis_name="c", subcore_axis_name="s")
_sc_params = pltpu.CompilerParams(use_tc_tiling_on_sc=False)


def _zero_accum(s, zbuf, accum):
  def zrow(i, _):
    zbuf[i, :] = jnp.zeros((D_HID,), jnp.float32)
    return ()
  lax.fori_loop(0, ZR, zrow, ())
  pltpu.sync_copy(zbuf, accum.at[pl.ds(s * ZR, ZR)])
  plsc.subcore_barrier()


def _flush_accum(c, s, accum, out_hbm):
  plsc.subcore_barrier()
  pltpu.sync_copy(accum.at[pl.ds(s * ZR, ZR)],
                  out_hbm.at[c, pl.ds(s * ZR, ZR)])


@functools.partial(
    pl.kernel,
    out_type=jax.ShapeDtypeStruct((NC, N_ACC, D_HID), jnp.float32),
    mesh=_mesh,
    compiler_params=_sc_params,
    scratch_types=[
        pltpu.VMEM((K, CH), jnp.int32),
        pltpu.VMEM((CH, D_HID), jnp.float32),
        pltpu.VMEM((ZR, D_HID), jnp.float32),
        pltpu.VMEM_SHARED((N_ACC, D_HID), jnp.float32),
        [pltpu.SemaphoreType.DMA] * NBUF,
    ],
)
def _sc_degree(dst_hbm, out_hbm, dst_v, ones_v, zbuf, accum, ssem):
  c = lax.axis_index("c")
  s = lax.axis_index("s")
  wid = s * NC + c
  _zero_accum(s, zbuf, accum)

  def orow(i, _):
    ones_v[i, :] = jnp.ones((D_HID,), jnp.float32)
    return ()
  lax.fori_loop(0, CH, orow, ())
  pltpu.sync_copy(dst_hbm.at[wid], dst_v)

  def body(jj, _):
    for r in range(NBUF):
      j = jj * NBUF + r

      @pl.when(jj > 0)
      def _():
        pltpu.make_async_copy(ones_v, accum.at[dst_v.at[j - NBUF]],
                              ssem[r]).wait()

      pltpu.async_copy(ones_v, accum.at[dst_v.at[j]], ssem[r], add=True)
    return ()

  lax.fori_loop(0, K // NBUF, body, ())
  for i in range(K - NBUF, K):
    pltpu.make_async_copy(ones_v, accum.at[dst_v.at[i]],
                          ssem[i % NBUF]).wait()
  _flush_accum(c, s, accum, out_hbm)


@functools.partial(
    pl.kernel,
    out_type=jax.ShapeDtypeStruct((NC, N_ACC, D_HID), jnp.float32),
    mesh=_mesh,
    compiler_params=_sc_params,
    scratch_types=[
        pltpu.VMEM((K, CH), jnp.int32),
        pltpu.VMEM((K, CH), jnp.int32),
        [pltpu.VMEM((CH, D_HID), jnp.float32)] * NBUF,
        pltpu.VMEM((ZR, D_HID), jnp.float32),
        pltpu.VMEM_SHARED((N_ACC, D_HID), jnp.float32),
        [pltpu.SemaphoreType.DMA] * NBUF,
        [pltpu.SemaphoreType.DMA] * NBUF,
    ],
)
def _sc_aggregate(table_hbm, src_hbm, dst_hbm, out_hbm,
                  src_v, dst_v, rows, zbuf, accum, gsem, ssem):
  c = lax.axis_index("c")
  s = lax.axis_index("s")
  wid = s * NC + c
  _zero_accum(s, zbuf, accum)
  pltpu.sync_copy(src_hbm.at[wid], src_v)
  pltpu.sync_copy(dst_hbm.at[wid], dst_v)

  for r in range(NINF):
    pltpu.async_copy(table_hbm.at[src_v.at[r]], rows[r], gsem[r])

  def body(jj, _):
    for r in range(NBUF):
      j = jj * NBUF + r
      rd = (r + NINF) % NBUF
      pltpu.make_async_copy(table_hbm.at[src_v.at[j]], rows[r], gsem[r]).wait()
      pltpu.async_copy(rows[r], accum.at[dst_v.at[j]], ssem[r], add=True)

      @pl.when(j >= NINF)
      def _():
        pltpu.make_async_copy(rows[rd], accum.at[dst_v.at[j - NINF]],
                              ssem[rd]).wait()

      @pl.when(j + NINF < K)
      def _():
        pltpu.async_copy(table_hbm.at[src_v.at[j + NINF]], rows[rd], gsem[rd])
    return ()

  lax.fori_loop(0, K // NBUF, body, ())
  for i in range(K - NINF, K):
    pltpu.make_async_copy(rows[i % NBUF], accum.at[dst_v.at[i]],
                          ssem[i % NBUF]).wait()
  _flush_accum(c, s, accum, out_hbm)


def _tc1_body(deg_ref, x_ref, w1_ref, dinv_ref, p_ref):
  deg = deg_ref[0, :N_NODES, :] + deg_ref[1, :N_NODES, :] + 1.0
  dinv = lax.rsqrt(deg)
  dinv_ref[...] = dinv
  p_ref[...] = jnp.dot(x_ref[...], w1_ref[...],
                       preferred_element_type=jnp.float32) * dinv


def _tc2_body(agg_ref, p_ref, dinv_ref, b1_ref, q_ref):
  dinv = dinv_ref[...]
  s = agg_ref[0, :N_NODES, :] + agg_ref[1, :N_NODES, :] + p_ref[...]
  h = jnp.maximum(dinv * s + b1_ref[...], 0.0)
  q_ref[...] = h * dinv


def _tc3_body(agg_ref, q_ref, dinv_ref, w2_ref, b2_ref, out_ref):
  s = agg_ref[0, :N_NODES, :] + agg_ref[1, :N_NODES, :] + q_ref[...]
  z = jnp.dot(dinv_ref[...] * s, w2_ref[...],
              preferred_element_type=jnp.float32) + b2_ref[...]
  m = jnp.max(z, axis=1, keepdims=True)
  zs = z - m
  out_ref[...] = zs - jnp.log(jnp.sum(jnp.exp(zs), axis=1, keepdims=True))


def kernel(x, edge_index, W1, b1, W2, b2):
  src = jnp.concatenate(
      [edge_index[0], jnp.zeros((E_PAD - N_EDGES,), jnp.int32)])
  dst = jnp.concatenate(
      [edge_index[1], jnp.full((E_PAD - N_EDGES,), N_NODES, jnp.int32)])
  src3 = src.reshape(NW, K, CH)
  dst3 = dst.reshape(NW, K, CH)

  deg16 = _sc_degree(dst3)

  dinv, p = pl.pallas_call(
      _tc1_body,
      out_shape=(
          jax.ShapeDtypeStruct((N_NODES, D_HID), jnp.float32),
          jax.ShapeDtypeStruct((N_NODES, D_HID), jnp.float32),
      ),
  )(deg16, x, W1)

  agg1 = _sc_aggregate(p, src3, dst3)

  q = pl.pallas_call(
      _tc2_body,
      out_shape=jax.ShapeDtypeStruct((N_NODES, D_HID), jnp.float32),
  )(agg1, p, dinv, b1.reshape(1, D_HID))

  agg2 = _sc_aggregate(q, src3, dst3)

  out = pl.pallas_call(
      _tc3_body,
      out_shape=jax.ShapeDtypeStruct((N_NODES, D_OUT), jnp.float32),
  )(agg2, q, dinv, W2, b2.reshape(1, D_OUT))
  return out


# speedup vs baseline: 1.0653x; 1.0047x over previous
import functools

import jax
import jax.numpy as jnp
from jax import lax
from jax.experimental import pallas as pl
from jax.experimental.pallas import tpu as pltpu
from jax.experimental.pallas import tpu_sc as plsc

N_NODES = 10000
N_EDGES = 320000
D_IN = 128
D_HID = 16
D_OUT = 5

NC = 2
NS = 16
NW = NC * NS
CH = 128
NBUF = 8
NINF = NBUF // 2
K = NBUF * (-(-N_EDGES // (NW * CH * NBUF)))
E_PAD = NW * K * CH
N_ACC = 10240
ZR = N_ACC // NS

_mesh = plsc.VectorSubcoreMesh(core_axis_name="c", subcore_axis_name="s")
_sc_params = pltpu.CompilerParams(use_tc_tiling_on_sc=False)


def _zero_accum(s, zbuf, accum):
  def zrow(i, _):
    zbuf[i, :] = jnp.zeros((D_HID,), jnp.float32)
    return ()
  lax.fori_loop(0, ZR, zrow, ())
  pltpu.sync_copy(zbuf, accum.at[pl.ds(s * ZR, ZR)])
  plsc.subcore_barrier()


def _flush_accum(c, s, accum, out_hbm):
  plsc.subcore_barrier()
  pltpu.sync_copy(accum.at[pl.ds(s * ZR, ZR)],
                  out_hbm.at[c, pl.ds(s * ZR, ZR)])


@functools.partial(
    pl.kernel,
    out_type=jax.ShapeDtypeStruct((NC, N_ACC, D_HID), jnp.float32),
    mesh=_mesh,
    compiler_params=_sc_params,
    scratch_types=[
        pltpu.VMEM((K, CH), jnp.int32),
        pltpu.VMEM((CH, D_HID), jnp.float32),
        pltpu.VMEM((ZR, D_HID), jnp.float32),
        pltpu.VMEM_SHARED((N_ACC, D_HID), jnp.float32),
    ],
)
def _sc_degree(dst_hbm, out_hbm, dst_v, ones_v, zbuf, accum):
  c = lax.axis_index("c")
  s = lax.axis_index("s")
  wid = s * NC + c
  _zero_accum(s, zbuf, accum)

  def orow(i, _):
    ones_v[i, :] = jnp.ones((D_HID,), jnp.float32)
    return ()
  lax.fori_loop(0, CH, orow, ())
  pltpu.sync_copy(dst_hbm.at[wid], dst_v)

  def body(j, _):
    pltpu.sync_copy(ones_v, accum.at[dst_v.at[j]], add=True)
    return ()
  lax.fori_loop(0, K, body, ())
  _flush_accum(c, s, accum, out_hbm)


@functools.partial(
    pl.kernel,
    out_type=jax.ShapeDtypeStruct((NC, N_ACC, D_HID), jnp.float32),
    mesh=_mesh,
    compiler_params=_sc_params,
    scratch_types=[
        pltpu.VMEM((K, CH), jnp.int32),
        pltpu.VMEM((K, CH), jnp.int32),
        [pltpu.VMEM((CH, D_HID), jnp.float32)] * NBUF,
        pltpu.VMEM((ZR, D_HID), jnp.float32),
        pltpu.VMEM_SHARED((N_ACC, D_HID), jnp.float32),
        [pltpu.SemaphoreType.DMA] * NBUF,
        [pltpu.SemaphoreType.DMA] * NBUF,
    ],
)
def _sc_aggregate(table_hbm, src_hbm, dst_hbm, out_hbm,
                  src_v, dst_v, rows, zbuf, accum, gsem, ssem):
  c = lax.axis_index("c")
  s = lax.axis_index("s")
  wid = s * NC + c
  _zero_accum(s, zbuf, accum)
  pltpu.sync_copy(src_hbm.at[wid], src_v)
  pltpu.sync_copy(dst_hbm.at[wid], dst_v)

  for r in range(NINF):
    pltpu.async_copy(table_hbm.at[src_v.at[r]], rows[r], gsem[r])

  def body(jj, _):
    for r in range(NBUF):
      j = jj * NBUF + r
      rd = (r + NINF) % NBUF
      pltpu.make_async_copy(table_hbm.at[src_v.at[j]], rows[r], gsem[r]).wait()
      pltpu.async_copy(rows[r], accum.at[dst_v.at[j]], ssem[r], add=True)

      @pl.when(j >= NINF)
      def _():
        pltpu.make_async_copy(rows[rd], accum.at[dst_v.at[j - NINF]],
                              ssem[rd]).wait()

      @pl.when(j + NINF < K)
      def _():
        pltpu.async_copy(table_hbm.at[src_v.at[j + NINF]], rows[rd], gsem[rd])
    return ()

  lax.fori_loop(0, K // NBUF, body, ())
  for i in range(K - NINF, K):
    pltpu.make_async_copy(rows[i % NBUF], accum.at[dst_v.at[i]],
                          ssem[i % NBUF]).wait()
  _flush_accum(c, s, accum, out_hbm)


def _tc1_body(deg_ref, x_ref, w1_ref, dinv_ref, p_ref):
  deg = deg_ref[0, :N_NODES, :] + deg_ref[1, :N_NODES, :] + 1.0
  dinv = lax.rsqrt(deg)
  dinv_ref[...] = dinv
  p_ref[...] = jnp.dot(x_ref[...], w1_ref[...],
                       preferred_element_type=jnp.float32) * dinv


def _tc2_body(agg_ref, p_ref, dinv_ref, b1_ref, q_ref):
  dinv = dinv_ref[...]
  s = agg_ref[0, :N_NODES, :] + agg_ref[1, :N_NODES, :] + p_ref[...]
  h = jnp.maximum(dinv * s + b1_ref[...], 0.0)
  q_ref[...] = h * dinv


def _tc3_body(agg_ref, q_ref, dinv_ref, w2_ref, b2_ref, out_ref):
  s = agg_ref[0, :N_NODES, :] + agg_ref[1, :N_NODES, :] + q_ref[...]
  z = jnp.dot(dinv_ref[...] * s, w2_ref[...],
              preferred_element_type=jnp.float32) + b2_ref[...]
  m = jnp.max(z, axis=1, keepdims=True)
  zs = z - m
  out_ref[...] = zs - jnp.log(jnp.sum(jnp.exp(zs), axis=1, keepdims=True))


def kernel(x, edge_index, W1, b1, W2, b2):
  src = jnp.concatenate(
      [edge_index[0], jnp.zeros((E_PAD - N_EDGES,), jnp.int32)])
  dst = jnp.concatenate(
      [edge_index[1], jnp.full((E_PAD - N_EDGES,), N_NODES, jnp.int32)])
  src3 = src.reshape(NW, K, CH)
  dst3 = dst.reshape(NW, K, CH)

  deg16 = _sc_degree(dst3)

  dinv, p = pl.pallas_call(
      _tc1_body,
      out_shape=(
          jax.ShapeDtypeStruct((N_NODES, D_HID), jnp.float32),
          jax.ShapeDtypeStruct((N_NODES, D_HID), jnp.float32),
      ),
  )(deg16, x, W1)

  agg1 = _sc_aggregate(p, src3, dst3)

  q = pl.pallas_call(
      _tc2_body,
      out_shape=jax.ShapeDtypeStruct((N_NODES, D_HID), jnp.float32),
  )(agg1, p, dinv, b1.reshape(1, D_HID))

  agg2 = _sc_aggregate(q, src3, dst3)

  out = pl.pallas_call(
      _tc3_body,
      out_shape=jax.ShapeDtypeStruct((N_NODES, D_OUT), jnp.float32),
  )(agg2, q, dinv, W2, b2.reshape(1, D_OUT))
  return out


# speedup vs baseline: 1.1030x; 1.0353x over previous
import functools

import jax
import jax.numpy as jnp
from jax import lax
from jax.experimental import pallas as pl
from jax.experimental.pallas import tpu as pltpu
from jax.experimental.pallas import tpu_sc as plsc

N_NODES = 10000
N_EDGES = 320000
D_IN = 128
D_HID = 16
D_OUT = 5

NC = 2
NS = 16
NW = NC * NS
CH = 128
NBUF = 8
NINF = NBUF // 2
K = NBUF * (-(-N_EDGES // (NW * CH * NBUF)))
E_PAD = NW * K * CH
N_ACC = 10240
ZR = N_ACC // NS

_mesh = plsc.VectorSubcoreMesh(core_axis_name="c", subcore_axis_name="s")
_sc_params = pltpu.CompilerParams(use_tc_tiling_on_sc=False)


def _zero_accum(s, zbuf, accum):
  def zrow(ii, _):
    for u in range(8):
      zbuf[ii * 8 + u, :] = jnp.zeros((D_HID,), jnp.float32)
    return ()
  lax.fori_loop(0, ZR // 8, zrow, ())
  pltpu.sync_copy(zbuf, accum.at[pl.ds(s * ZR, ZR)])
  plsc.subcore_barrier()


def _flush_accum(c, s, accum, out_hbm):
  plsc.subcore_barrier()
  pltpu.sync_copy(accum.at[pl.ds(s * ZR, ZR)],
                  out_hbm.at[c, pl.ds(s * ZR, ZR)])


@functools.partial(
    pl.kernel,
    out_type=jax.ShapeDtypeStruct((NC, N_ACC, D_HID), jnp.float32),
    mesh=_mesh,
    compiler_params=_sc_params,
    scratch_types=[
        pltpu.VMEM((K, CH), jnp.int32),
        pltpu.VMEM((CH, D_HID), jnp.float32),
        pltpu.VMEM((ZR, D_HID), jnp.float32),
        pltpu.VMEM_SHARED((N_ACC, D_HID), jnp.float32),
    ],
)
def _sc_degree(dst_hbm, out_hbm, dst_v, ones_v, zbuf, accum):
  c = lax.axis_index("c")
  s = lax.axis_index("s")
  wid = s * NC + c
  _zero_accum(s, zbuf, accum)

  def orow(i, _):
    ones_v[i, :] = jnp.ones((D_HID,), jnp.float32)
    return ()
  lax.fori_loop(0, CH, orow, ())
  pltpu.sync_copy(dst_hbm.at[wid], dst_v)

  def body(j, _):
    pltpu.sync_copy(ones_v, accum.at[dst_v.at[j]], add=True)
    return ()
  lax.fori_loop(0, K, body, ())
  _flush_accum(c, s, accum, out_hbm)


@functools.partial(
    pl.kernel,
    out_type=jax.ShapeDtypeStruct((NC, N_ACC, D_HID), jnp.float32),
    mesh=_mesh,
    compiler_params=_sc_params,
    scratch_types=[
        pltpu.VMEM((K, CH), jnp.int32),
        pltpu.VMEM((K, CH), jnp.int32),
        [pltpu.VMEM((CH, D_HID), jnp.float32)] * NBUF,
        pltpu.VMEM((ZR, D_HID), jnp.float32),
        pltpu.VMEM_SHARED((N_ACC, D_HID), jnp.float32),
        [pltpu.SemaphoreType.DMA] * NBUF,
        [pltpu.SemaphoreType.DMA] * NBUF,
    ],
)
def _sc_aggregate(table_hbm, src_hbm, dst_hbm, out_hbm,
                  src_v, dst_v, rows, zbuf, accum, gsem, ssem):
  c = lax.axis_index("c")
  s = lax.axis_index("s")
  wid = s * NC + c
  _zero_accum(s, zbuf, accum)
  pltpu.sync_copy(src_hbm.at[wid], src_v)
  pltpu.sync_copy(dst_hbm.at[wid], dst_v)

  for r in range(NINF):
    pltpu.async_copy(table_hbm.at[src_v.at[r]], rows[r], gsem[r])

  def body(jj, _):
    for r in range(NBUF):
      j = jj * NBUF + r
      rd = (r + NINF) % NBUF
      pltpu.make_async_copy(table_hbm.at[src_v.at[j]], rows[r], gsem[r]).wait()
      pltpu.async_copy(rows[r], accum.at[dst_v.at[j]], ssem[r], add=True)

      @pl.when(j >= NINF)
      def _():
        pltpu.make_async_copy(rows[rd], accum.at[dst_v.at[j - NINF]],
                              ssem[rd]).wait()

      @pl.when(j + NINF < K)
      def _():
        pltpu.async_copy(table_hbm.at[src_v.at[j + NINF]], rows[rd], gsem[rd])
    return ()

  lax.fori_loop(0, K // NBUF, body, ())
  for i in range(K - NINF, K):
    pltpu.make_async_copy(rows[i % NBUF], accum.at[dst_v.at[i]],
                          ssem[i % NBUF]).wait()
  _flush_accum(c, s, accum, out_hbm)


def _tc1_body(deg_ref, x_ref, w1_ref, dinv_ref, p_ref):
  deg = deg_ref[0, :N_NODES, :] + deg_ref[1, :N_NODES, :] + 1.0
  dinv = lax.rsqrt(deg)
  dinv_ref[...] = dinv
  p_ref[...] = jnp.dot(x_ref[...], w1_ref[...],
                       preferred_element_type=jnp.float32) * dinv


def _tc2_body(agg_ref, p_ref, dinv_ref, b1_ref, q_ref):
  dinv = dinv_ref[...]
  s = agg_ref[0, :N_NODES, :] + agg_ref[1, :N_NODES, :] + p_ref[...]
  h = jnp.maximum(dinv * s + b1_ref[...], 0.0)
  q_ref[...] = h * dinv


def _tc3_body(agg_ref, q_ref, dinv_ref, w2_ref, b2_ref, out_ref):
  s = agg_ref[0, :N_NODES, :] + agg_ref[1, :N_NODES, :] + q_ref[...]
  z = jnp.dot(dinv_ref[...] * s, w2_ref[...],
              preferred_element_type=jnp.float32) + b2_ref[...]
  m = jnp.max(z, axis=1, keepdims=True)
  zs = z - m
  out_ref[...] = zs - jnp.log(jnp.sum(jnp.exp(zs), axis=1, keepdims=True))


def kernel(x, edge_index, W1, b1, W2, b2):
  src = jnp.concatenate(
      [edge_index[0], jnp.zeros((E_PAD - N_EDGES,), jnp.int32)])
  dst = jnp.concatenate(
      [edge_index[1], jnp.full((E_PAD - N_EDGES,), N_NODES, jnp.int32)])
  src3 = src.reshape(NW, K, CH)
  dst3 = dst.reshape(NW, K, CH)

  deg16 = _sc_degree(dst3)

  dinv, p = pl.pallas_call(
      _tc1_body,
      out_shape=(
          jax.ShapeDtypeStruct((N_NODES, D_HID), jnp.float32),
          jax.ShapeDtypeStruct((N_NODES, D_HID), jnp.float32),
      ),
  )(deg16, x, W1)

  agg1 = _sc_aggregate(p, src3, dst3)

  q = pl.pallas_call(
      _tc2_body,
      out_shape=jax.ShapeDtypeStruct((N_NODES, D_HID), jnp.float32),
  )(agg1, p, dinv, b1.reshape(1, D_HID))

  agg2 = _sc_aggregate(q, src3, dst3)

  out = pl.pallas_call(
      _tc3_body,
      out_shape=jax.ShapeDtypeStruct((N_NODES, D_OUT), jnp.float32),
  )(agg2, q, dinv, W2, b2.reshape(1, D_OUT))
  return out


# speedup vs baseline: 1.1034x; 1.0004x over previous
import functools

import jax
import jax.numpy as jnp
from jax import lax
from jax.experimental import pallas as pl
from jax.experimental.pallas import tpu as pltpu
from jax.experimental.pallas import tpu_sc as plsc

N_NODES = 10000
N_EDGES = 320000
D_IN = 128
D_HID = 16
D_OUT = 5

NC = 2
NS = 16
NW = NC * NS
CH = 128
NBUF = 8
NINF = NBUF // 2
K = NBUF * (-(-N_EDGES // (NW * CH * NBUF)))
E_PAD = NW * K * CH
N_ACC = 10240
ZR = N_ACC // NS

_mesh = plsc.VectorSubcoreMesh(core_axis_name="c", subcore_axis_name="s")
_sc_params = pltpu.CompilerParams(use_tc_tiling_on_sc=False)


def _zero_accum(s, zbuf, accum):
  def zrow(ii, _):
    for u in range(8):
      zbuf[ii * 8 + u, :] = jnp.zeros((D_HID,), jnp.float32)
    return ()
  lax.fori_loop(0, ZR // 8, zrow, ())
  pltpu.sync_copy(zbuf, accum.at[pl.ds(s * ZR, ZR)])
  plsc.subcore_barrier()


def _flush_accum(c, s, accum, out_hbm):
  plsc.subcore_barrier()
  pltpu.sync_copy(accum.at[pl.ds(s * ZR, ZR)],
                  out_hbm.at[c, pl.ds(s * ZR, ZR)])


@functools.partial(
    pl.kernel,
    out_type=jax.ShapeDtypeStruct((NC, N_ACC, D_HID), jnp.float32),
    mesh=_mesh,
    compiler_params=_sc_params,
    scratch_types=[
        pltpu.VMEM((K, CH), jnp.int32),
        pltpu.VMEM((CH, D_HID), jnp.float32),
        pltpu.VMEM((ZR, D_HID), jnp.float32),
        pltpu.VMEM_SHARED((N_ACC, D_HID), jnp.float32),
    ],
)
def _sc_degree(dst_hbm, out_hbm, dst_v, ones_v, zbuf, accum):
  c = lax.axis_index("c")
  s = lax.axis_index("s")
  wid = s * NC + c
  _zero_accum(s, zbuf, accum)

  def orow(ii, _):
    for u in range(8):
      ones_v[ii * 8 + u, :] = jnp.ones((D_HID,), jnp.float32)
    return ()
  lax.fori_loop(0, CH // 8, orow, ())
  pltpu.sync_copy(dst_hbm.at[wid], dst_v)

  def body(j, _):
    pltpu.sync_copy(ones_v, accum.at[dst_v.at[j]], add=True)
    return ()
  lax.fori_loop(0, K, body, ())
  _flush_accum(c, s, accum, out_hbm)


@functools.partial(
    pl.kernel,
    out_type=jax.ShapeDtypeStruct((NC, N_ACC, D_HID), jnp.float32),
    mesh=_mesh,
    compiler_params=_sc_params,
    scratch_types=[
        pltpu.VMEM((K, CH), jnp.int32),
        pltpu.VMEM((K, CH), jnp.int32),
        [pltpu.VMEM((CH, D_HID), jnp.float32)] * NBUF,
        pltpu.VMEM((ZR, D_HID), jnp.float32),
        pltpu.VMEM_SHARED((N_ACC, D_HID), jnp.float32),
        [pltpu.SemaphoreType.DMA] * NBUF,
        [pltpu.SemaphoreType.DMA] * NBUF,
    ],
)
def _sc_aggregate(table_hbm, src_hbm, dst_hbm, out_hbm,
                  src_v, dst_v, rows, zbuf, accum, gsem, ssem):
  c = lax.axis_index("c")
  s = lax.axis_index("s")
  wid = s * NC + c
  _zero_accum(s, zbuf, accum)
  pltpu.sync_copy(src_hbm.at[wid], src_v)
  pltpu.sync_copy(dst_hbm.at[wid], dst_v)

  for r in range(NINF):
    pltpu.async_copy(table_hbm.at[src_v.at[r]], rows[r], gsem[r])

  def body(jj, _):
    for r in range(NBUF):
      j = jj * NBUF + r
      rd = (r + NINF) % NBUF
      pltpu.make_async_copy(table_hbm.at[src_v.at[j]], rows[r], gsem[r]).wait()
      pltpu.async_copy(rows[r], accum.at[dst_v.at[j]], ssem[r], add=True)

      @pl.when(j >= NINF)
      def _():
        pltpu.make_async_copy(rows[rd], accum.at[dst_v.at[j - NINF]],
                              ssem[rd]).wait()

      @pl.when(j + NINF < K)
      def _():
        pltpu.async_copy(table_hbm.at[src_v.at[j + NINF]], rows[rd], gsem[rd])
    return ()

  lax.fori_loop(0, K // NBUF, body, ())
  for i in range(K - NINF, K):
    pltpu.make_async_copy(rows[i % NBUF], accum.at[dst_v.at[i]],
                          ssem[i % NBUF]).wait()
  _flush_accum(c, s, accum, out_hbm)


def _tc1_body(deg_ref, x_ref, w1_ref, dinv_ref, p_ref):
  deg = deg_ref[0, :N_NODES, :] + deg_ref[1, :N_NODES, :] + 1.0
  dinv = lax.rsqrt(deg)
  dinv_ref[...] = dinv
  p_ref[...] = jnp.dot(x_ref[...], w1_ref[...],
                       preferred_element_type=jnp.float32) * dinv


def _tc2_body(agg_ref, p_ref, dinv_ref, b1_ref, q_ref):
  dinv = dinv_ref[...]
  s = agg_ref[0, :N_NODES, :] + agg_ref[1, :N_NODES, :] + p_ref[...]
  h = jnp.maximum(dinv * s + b1_ref[...], 0.0)
  q_ref[...] = h * dinv


def _tc3_body(agg_ref, q_ref, dinv_ref, w2_ref, b2_ref, out_ref):
  s = agg_ref[0, :N_NODES, :] + agg_ref[1, :N_NODES, :] + q_ref[...]
  z = jnp.dot(dinv_ref[...] * s, w2_ref[...],
              preferred_element_type=jnp.float32) + b2_ref[...]
  m = jnp.max(z, axis=1, keepdims=True)
  zs = z - m
  out_ref[...] = zs - jnp.log(jnp.sum(jnp.exp(zs), axis=1, keepdims=True))


def kernel(x, edge_index, W1, b1, W2, b2):
  src = jnp.concatenate(
      [edge_index[0], jnp.zeros((E_PAD - N_EDGES,), jnp.int32)])
  dst = jnp.concatenate(
      [edge_index[1], jnp.full((E_PAD - N_EDGES,), N_NODES, jnp.int32)])
  src3 = src.reshape(NW, K, CH)
  dst3 = dst.reshape(NW, K, CH)

  deg16 = _sc_degree(dst3)

  dinv, p = pl.pallas_call(
      _tc1_body,
      out_shape=(
          jax.ShapeDtypeStruct((N_NODES, D_HID), jnp.float32),
          jax.ShapeDtypeStruct((N_NODES, D_HID), jnp.float32),
      ),
  )(deg16, x, W1)

  agg1 = _sc_aggregate(p, src3, dst3)

  q = pl.pallas_call(
      _tc2_body,
      out_shape=jax.ShapeDtypeStruct((N_NODES, D_HID), jnp.float32),
  )(agg1, p, dinv, b1.reshape(1, D_HID))

  agg2 = _sc_aggregate(q, src3, dst3)

  out = pl.pallas_call(
      _tc3_body,
      out_shape=jax.ShapeDtypeStruct((N_NODES, D_OUT), jnp.float32),
  )(agg2, q, dinv, W2, b2.reshape(1, D_OUT))
  return out
